# baseline (speedup 1.0000x reference)
import functools

import jax
import jax.numpy as jnp
from jax import lax
from jax.experimental import pallas as pl
from jax.experimental.pallas import tpu as pltpu
from jax.experimental.pallas import tpu_sc as plsc

V = 1000000
D = 32
B = 16384
NC = 2
NS = 16
NW = NC * NS
BPW = B // NW
NCHUNK = 4
CHUNK = BPW // NCHUNK

_LN2 = 0.6931471805599453
_LN100 = 4.605170185988092

_mesh = plsc.VectorSubcoreMesh(core_axis_name="c", subcore_axis_name="s")


@functools.partial(
    pl.kernel,
    mesh=_mesh,
    compiler_params=pltpu.CompilerParams(
        needs_layout_passes=False, use_tc_tiling_on_sc=False),
    out_type=jax.ShapeDtypeStruct((B,), jnp.float32),
    scratch_types=[
        pltpu.VMEM((NCHUNK, CHUNK), jnp.int32),
        pltpu.VMEM((NCHUNK, CHUNK), jnp.int32),
        pltpu.VMEM((D, BPW), jnp.float32),
        pltpu.VMEM((D, BPW), jnp.float32),
        pltpu.VMEM((BPW,), jnp.float32),
        pltpu.VMEM((BPW,), jnp.float32),
        pltpu.VMEM((BPW,), jnp.float32),
        pltpu.VMEM((BPW,), jnp.float32),
        pltpu.SemaphoreType.DMA,
        pltpu.SemaphoreType.DMA,
    ],
)
def _glove_sc(wi_hbm, wj_hbm, y_hbm, fc_hbm, fx_hbm, bc_hbm, bx_hbm,
              out_hbm, idx_i, idx_j, e_i, e_j, bi, bj, yv, outv,
              sem, bsem):
    wid = lax.axis_index("s") * NC + lax.axis_index("c")
    base = wid * BPW

    pltpu.sync_copy(wi_hbm.at[pl.ds(wid * NCHUNK, NCHUNK)], idx_i)
    pltpu.sync_copy(wj_hbm.at[pl.ds(wid * NCHUNK, NCHUNK)], idx_j)
    pltpu.sync_copy(y_hbm.at[pl.ds(base, BPW)], yv)

    bias_copies = []
    for k in range(NCHUNK):
        sl = pl.ds(k * CHUNK, CHUNK)
        bias_copies.append(
            pltpu.async_copy(bc_hbm.at[idx_i.at[k]], bi.at[sl], bsem))
        bias_copies.append(
            pltpu.async_copy(bx_hbm.at[idx_j.at[k]], bj.at[sl], bsem))

    def issue(g, carry):
        k = g // (CHUNK // 16)
        c = (g % (CHUNK // 16)) * 16
        off = g * 16
        vi = idx_i[k, pl.ds(c, 16)]
        vj = idx_j[k, pl.ds(c, 16)]
        for d in range(D):
            pltpu.async_copy(
                fc_hbm.at[vi + d * V], e_i.at[d, pl.ds(off, 16)], sem)
            pltpu.async_copy(
                fx_hbm.at[vj + d * V], e_j.at[d, pl.ds(off, 16)], sem)
        return carry

    lax.fori_loop(0, BPW // 16, issue, 0)
    for c in bias_copies:
        c.wait()
    pltpu.make_async_copy(fc_hbm.at[pl.ds(0, D * BPW)], e_i, sem).wait()
    pltpu.make_async_copy(fx_hbm.at[pl.ds(0, D * BPW)], e_j, sem).wait()

    def group(g, carry):
        off = g * 16
        acc = jnp.zeros((16,), jnp.float32)
        for d in range(D):
            acc = acc + e_i[d, pl.ds(off, 16)] * e_j[d, pl.ds(off, 16)]
        pred = acc + bi[pl.ds(off, 16)] + bj[pl.ds(off, 16)]
        y = yv[pl.ds(off, 16)]
        bits = plsc.bitcast(y, jnp.int32)
        ex = (bits >> 23) - 127
        m = plsc.bitcast((bits & 0x007FFFFF) | 0x3F800000, jnp.float32)
        t = (m - 1.0) / (m + 1.0)
        t2 = t * t
        lnm = 2.0 * t * (1.0 + t2 * (1.0 / 3.0 + t2 * (0.2 + t2 * (1.0 / 7.0))))
        lny = ex.astype(jnp.float32) * _LN2 + lnm
        w = jnp.minimum(jnp.exp(0.75 * (lny - _LN100)), 1.0)
        r = pred - lny
        outv[pl.ds(off, 16)] = w * r * r
        return carry

    lax.fori_loop(0, BPW // 16, group, 0)
    pltpu.sync_copy(outv, out_hbm.at[pl.ds(base, BPW)])


def kernel(w_i, w_j, y_true, W_center, W_context, b_center, b_context):
    wi = w_i.astype(jnp.int32).reshape(NW * NCHUNK, CHUNK)
    wj = w_j.astype(jnp.int32).reshape(NW * NCHUNK, CHUNK)
    fc = jnp.ravel(W_center.T)
    fx = jnp.ravel(W_context.T)
    return _glove_sc(wi, wj, y_true, fc, fx, b_center, b_context)


# speedup vs baseline: 1.0024x; 1.0024x over previous
import functools

import jax
import jax.numpy as jnp
from jax import lax
from jax.experimental import pallas as pl
from jax.experimental.pallas import tpu as pltpu
from jax.experimental.pallas import tpu_sc as plsc

V = 1000000
D = 32
B = 16384
NC = 2
NS = 16
NW = NC * NS
BPW = B // NW
NCHUNK = 4
CHUNK = BPW // NCHUNK

_LN2 = 0.6931471805599453
_LN100 = 4.605170185988092

_mesh = plsc.VectorSubcoreMesh(core_axis_name="c", subcore_axis_name="s")


@functools.partial(
    pl.kernel,
    mesh=_mesh,
    compiler_params=pltpu.CompilerParams(
        needs_layout_passes=False, use_tc_tiling_on_sc=False),
    out_type=jax.ShapeDtypeStruct((B,), jnp.float32),
    scratch_types=[
        pltpu.VMEM((NCHUNK, CHUNK), jnp.int32),
        pltpu.VMEM((NCHUNK, CHUNK), jnp.int32),
        pltpu.VMEM((D, BPW), jnp.float32),
        pltpu.VMEM((D, BPW), jnp.float32),
        pltpu.VMEM((BPW,), jnp.float32),
        pltpu.VMEM((BPW,), jnp.float32),
        pltpu.VMEM((BPW,), jnp.float32),
        pltpu.VMEM((BPW,), jnp.float32),
        pltpu.SemaphoreType.DMA,
        pltpu.SemaphoreType.DMA,
    ],
)
def _glove_sc(wi_hbm, wj_hbm, y_hbm, fc_hbm, fx_hbm, bc_hbm, bx_hbm,
              out_hbm, idx_i, idx_j, e_i, e_j, bi, bj, yv, outv,
              sem, bsem):
    wid = lax.axis_index("s") * NC + lax.axis_index("c")
    base = wid * BPW

    pltpu.sync_copy(wi_hbm.at[pl.ds(wid * NCHUNK, NCHUNK)], idx_i)
    pltpu.sync_copy(wj_hbm.at[pl.ds(wid * NCHUNK, NCHUNK)], idx_j)
    pltpu.sync_copy(y_hbm.at[pl.ds(base, BPW)], yv)

    bias_copies = []
    for k in range(NCHUNK):
        sl = pl.ds(k * CHUNK, CHUNK)
        bias_copies.append(
            pltpu.async_copy(bc_hbm.at[idx_i.at[k]], bi.at[sl], bsem))
        bias_copies.append(
            pltpu.async_copy(bx_hbm.at[idx_j.at[k]], bj.at[sl], bsem))

    def issue(d, carry):
        for k in range(NCHUNK):
            sl = pl.ds(k * CHUNK, CHUNK)
            pltpu.async_copy(
                fc_hbm.at[d].at[idx_i.at[k]], e_i.at[d].at[sl], sem)
            pltpu.async_copy(
                fx_hbm.at[d].at[idx_j.at[k]], e_j.at[d].at[sl], sem)
        return carry

    lax.fori_loop(0, D, issue, 0)
    for c in bias_copies:
        c.wait()
    pltpu.make_async_copy(fc_hbm.at[pl.ds(0, D), pl.ds(0, BPW)], e_i,
                          sem).wait()
    pltpu.make_async_copy(fx_hbm.at[pl.ds(0, D), pl.ds(0, BPW)], e_j,
                          sem).wait()

    def group(g, carry):
        off = g * 16
        acc = jnp.zeros((16,), jnp.float32)
        for d in range(D):
            acc = acc + e_i[d, pl.ds(off, 16)] * e_j[d, pl.ds(off, 16)]
        pred = acc + bi[pl.ds(off, 16)] + bj[pl.ds(off, 16)]
        y = yv[pl.ds(off, 16)]
        bits = plsc.bitcast(y, jnp.int32)
        ex = (bits >> 23) - 127
        m = plsc.bitcast((bits & 0x007FFFFF) | 0x3F800000, jnp.float32)
        t = (m - 1.0) / (m + 1.0)
        t2 = t * t
        lnm = 2.0 * t * (1.0 + t2 * (1.0 / 3.0 + t2 * (0.2 + t2 * (1.0 / 7.0))))
        lny = ex.astype(jnp.float32) * _LN2 + lnm
        w = jnp.minimum(jnp.exp(0.75 * (lny - _LN100)), 1.0)
        r = pred - lny
        outv[pl.ds(off, 16)] = w * r * r
        return carry

    lax.fori_loop(0, BPW // 16, group, 0)
    pltpu.sync_copy(outv, out_hbm.at[pl.ds(base, BPW)])


def kernel(w_i, w_j, y_true, W_center, W_context, b_center, b_context):
    wi = w_i.astype(jnp.int32).reshape(NW * NCHUNK, CHUNK)
    wj = w_j.astype(jnp.int32).reshape(NW * NCHUNK, CHUNK)
    return _glove_sc(wi, wj, y_true, W_center.T, W_context.T,
                     b_center, b_context)


# speedup vs baseline: 4.8185x; 4.8069x over previous
import functools

import jax
import jax.numpy as jnp
from jax import lax
from jax.experimental import pallas as pl
from jax.experimental.pallas import tpu as pltpu
from jax.experimental.pallas import tpu_sc as plsc

V = 1000000
D = 32
B = 16384
NC = 2
NS = 16
NW = NC * NS
BPW = B // NW
NCHUNK = 4
CHUNK = BPW // NCHUNK

_LN2 = 0.6931471805599453
_LN100 = 4.605170185988092

_mesh = plsc.VectorSubcoreMesh(core_axis_name="c", subcore_axis_name="s")


@functools.partial(
    pl.kernel,
    mesh=_mesh,
    compiler_params=pltpu.CompilerParams(
        needs_layout_passes=False, use_tc_tiling_on_sc=False),
    out_type=jax.ShapeDtypeStruct((B,), jnp.float32),
    scratch_types=[
        pltpu.VMEM((NCHUNK, CHUNK), jnp.int32),
        pltpu.VMEM((NCHUNK, CHUNK), jnp.int32),
        pltpu.VMEM((BPW, D), jnp.bfloat16),
        pltpu.VMEM((BPW, D), jnp.bfloat16),
        pltpu.VMEM((BPW, D), jnp.float32),
        pltpu.VMEM((BPW, D), jnp.float32),
        pltpu.VMEM((BPW,), jnp.float32),
        pltpu.VMEM((BPW,), jnp.float32),
        pltpu.VMEM((BPW,), jnp.float32),
        pltpu.VMEM((BPW,), jnp.float32),
        pltpu.SemaphoreType.DMA,
    ],
)
def _glove_sc(wi_hbm, wj_hbm, y_hbm, wc_hbm, wx_hbm, bc_hbm, bx_hbm,
              out_hbm, idx_i, idx_j, e_ib, e_jb, e_i, e_j,
              bi, bj, yv, outv, sem):
    wid = lax.axis_index("s") * NC + lax.axis_index("c")
    base = wid * BPW

    pltpu.sync_copy(wi_hbm.at[pl.ds(wid * NCHUNK, NCHUNK)], idx_i)
    pltpu.sync_copy(wj_hbm.at[pl.ds(wid * NCHUNK, NCHUNK)], idx_j)
    pltpu.sync_copy(y_hbm.at[pl.ds(base, BPW)], yv)

    copies = []
    for k in range(NCHUNK):
        sl = pl.ds(k * CHUNK, CHUNK)
        copies.append(pltpu.async_copy(wc_hbm.at[idx_i.at[k]], e_ib.at[sl], sem))
        copies.append(pltpu.async_copy(wx_hbm.at[idx_j.at[k]], e_jb.at[sl], sem))
        copies.append(pltpu.async_copy(bc_hbm.at[idx_i.at[k]], bi.at[sl], sem))
        copies.append(pltpu.async_copy(bx_hbm.at[idx_j.at[k]], bj.at[sl], sem))
    for c in copies:
        c.wait()

    def widen(r, carry):
        lo_i, hi_i = plsc.unpack(e_ib[r, :], format=plsc.PackFormat.INTERLEAVED)
        e_i[r, pl.ds(0, 16)] = lo_i
        e_i[r, pl.ds(16, 16)] = hi_i
        lo_j, hi_j = plsc.unpack(e_jb[r, :], format=plsc.PackFormat.INTERLEAVED)
        e_j[r, pl.ds(0, 16)] = lo_j
        e_j[r, pl.ds(16, 16)] = hi_j
        return carry

    lax.fori_loop(0, BPW, widen, 0)

    lane = lax.iota(jnp.int32, 16)

    def group(g, carry):
        rows = g * 16 + lane
        acc = jnp.zeros((16,), jnp.float32)
        for d in range(D):
            col = jnp.full((16,), d, jnp.int32)
            acc = acc + plsc.load_gather(e_i, [rows, col]) * \
                plsc.load_gather(e_j, [rows, col])
        off = g * 16
        pred = acc + bi[pl.ds(off, 16)] + bj[pl.ds(off, 16)]
        y = yv[pl.ds(off, 16)]
        bits = plsc.bitcast(y, jnp.int32)
        ex = (bits >> 23) - 127
        m = plsc.bitcast((bits & 0x007FFFFF) | 0x3F800000, jnp.float32)
        t = (m - 1.0) / (m + 1.0)
        t2 = t * t
        lnm = 2.0 * t * (1.0 + t2 * (1.0 / 3.0 + t2 * (0.2 + t2 * (1.0 / 7.0))))
        lny = ex.astype(jnp.float32) * _LN2 + lnm
        w = jnp.minimum(jnp.exp(0.75 * (lny - _LN100)), 1.0)
        r = pred - lny
        outv[pl.ds(off, 16)] = w * r * r
        return carry

    lax.fori_loop(0, BPW // 16, group, 0)
    pltpu.sync_copy(outv, out_hbm.at[pl.ds(base, BPW)])


def kernel(w_i, w_j, y_true, W_center, W_context, b_center, b_context):
    wi = w_i.astype(jnp.int32).reshape(NW * NCHUNK, CHUNK)
    wj = w_j.astype(jnp.int32).reshape(NW * NCHUNK, CHUNK)
    return _glove_sc(wi, wj, y_true,
                     W_center.astype(jnp.bfloat16),
                     W_context.astype(jnp.bfloat16),
                     b_center, b_context)


# speedup vs baseline: 5.6423x; 1.1710x over previous
import functools

import jax
import jax.numpy as jnp
from jax import lax
from jax.experimental import pallas as pl
from jax.experimental.pallas import tpu as pltpu
from jax.experimental.pallas import tpu_sc as plsc

V = 1000000
D = 32
B = 16384
NC = 2
NS = 16
NW = NC * NS
BPW = B // NW
NCHUNK = 4
CHUNK = BPW // NCHUNK

_LN2 = 0.6931471805599453
_LN100 = 4.605170185988092

_mesh = plsc.VectorSubcoreMesh(core_axis_name="c", subcore_axis_name="s")


@functools.partial(
    pl.kernel,
    mesh=_mesh,
    compiler_params=pltpu.CompilerParams(
        needs_layout_passes=False, use_tc_tiling_on_sc=False),
    out_type=jax.ShapeDtypeStruct((B,), jnp.float32),
    scratch_types=[
        pltpu.VMEM((NCHUNK, CHUNK), jnp.int32),
        pltpu.VMEM((NCHUNK, CHUNK), jnp.int32),
        pltpu.VMEM((BPW, D), jnp.float32),
        pltpu.VMEM((BPW, D), jnp.float32),
        pltpu.VMEM((BPW,), jnp.float32),
        pltpu.VMEM((BPW,), jnp.float32),
        pltpu.VMEM((BPW,), jnp.float32),
        pltpu.VMEM((BPW,), jnp.float32),
        pltpu.SemaphoreType.DMA,
    ],
)
def _glove_sc(wi_hbm, wj_hbm, y_hbm, wc_hbm, wx_hbm, bc_hbm, bx_hbm,
              out_hbm, idx_i, idx_j, e_i, e_j, bi, bj, yv, outv, sem):
    wid = lax.axis_index("s") * NC + lax.axis_index("c")
    base = wid * BPW

    pltpu.sync_copy(wi_hbm.at[pl.ds(wid * NCHUNK, NCHUNK)], idx_i)
    pltpu.sync_copy(wj_hbm.at[pl.ds(wid * NCHUNK, NCHUNK)], idx_j)
    pltpu.sync_copy(y_hbm.at[pl.ds(base, BPW)], yv)

    copies = []
    for k in range(NCHUNK):
        sl = pl.ds(k * CHUNK, CHUNK)
        copies.append(pltpu.async_copy(wc_hbm.at[idx_i.at[k]], e_i.at[sl], sem))
        copies.append(pltpu.async_copy(wx_hbm.at[idx_j.at[k]], e_j.at[sl], sem))
        copies.append(pltpu.async_copy(bc_hbm.at[idx_i.at[k]], bi.at[sl], sem))
        copies.append(pltpu.async_copy(bx_hbm.at[idx_j.at[k]], bj.at[sl], sem))
    for c in copies:
        c.wait()

    lane = lax.iota(jnp.int32, 16)

    def group(g, carry):
        rows = g * 16 + lane
        acc = jnp.zeros((16,), jnp.float32)
        for d in range(D):
            col = jnp.full((16,), d, jnp.int32)
            acc = acc + plsc.load_gather(e_i, [rows, col]) * \
                plsc.load_gather(e_j, [rows, col])
        off = g * 16
        pred = acc + bi[pl.ds(off, 16)] + bj[pl.ds(off, 16)]
        y = yv[pl.ds(off, 16)]
        bits = plsc.bitcast(y, jnp.int32)
        ex = (bits >> 23) - 127
        m = plsc.bitcast((bits & 0x007FFFFF) | 0x3F800000, jnp.float32)
        t = (m - 1.0) / (m + 1.0)
        t2 = t * t
        lnm = 2.0 * t * (1.0 + t2 * (1.0 / 3.0 + t2 * (0.2 + t2 * (1.0 / 7.0))))
        lny = ex.astype(jnp.float32) * _LN2 + lnm
        w = jnp.minimum(jnp.exp(0.75 * (lny - _LN100)), 1.0)
        r = pred - lny
        outv[pl.ds(off, 16)] = w * r * r
        return carry

    lax.fori_loop(0, BPW // 16, group, 0)
    pltpu.sync_copy(outv, out_hbm.at[pl.ds(base, BPW)])


def kernel(w_i, w_j, y_true, W_center, W_context, b_center, b_context):
    wi = w_i.astype(jnp.int32).reshape(NW * NCHUNK, CHUNK)
    wj = w_j.astype(jnp.int32).reshape(NW * NCHUNK, CHUNK)
    return _glove_sc(wi, wj, y_true, W_center, W_context, b_center, b_context)
